# baseline (speedup 1.0000x reference)
import functools

import jax
import jax.numpy as jnp
from jax import lax
from jax.experimental import pallas as pl
from jax.experimental.pallas import tpu as pltpu
from jax.experimental.pallas import tpu_sc as plsc

_B, _C, _H, _W = 4, 256, 128, 128
_S = _H * _W
_SCALE = 0.125
_NW = 32
_WPB = _NW // _B
_CPW = _C // _WPB
_NCH = 4
_NGRP = _CPW // _NCH
_CHUNK = 2048
_NCHUNK = _S // _CHUNK
_TOTCHUNK = _NGRP * _NCHUNK

_mesh = plsc.VectorSubcoreMesh(core_axis_name="c", subcore_axis_name="s")


@functools.partial(
    pl.kernel,
    out_type=jax.ShapeDtypeStruct((_B * _C * _S,), jnp.float32),
    mesh=_mesh,
    compiler_params=pltpu.CompilerParams(needs_layout_passes=False),
    scratch_types=[
        [pltpu.VMEM((_S,), jnp.float32) for _ in range(_NCH)],
        pltpu.VMEM((2 * 5 * _CHUNK,), jnp.float32),
        pltpu.VMEM((3, _NCH, _CHUNK), jnp.float32),
        pltpu.SemaphoreType.DMA,
        pltpu.SemaphoreType.DMA,
        pltpu.SemaphoreType.DMA,
        pltpu.SemaphoreType.DMA,
    ],
)
def _fr_kernel(feats, rbb, out, planes, rbbuf, outbuf, sem_pl, sem_rb,
               sem_out, sem_fi):
    cid = lax.axis_index("c")
    sid = lax.axis_index("s")
    wid = sid * 2 + cid
    b = wid // _WPB
    ch0 = (wid % _WPB) * _CPW

    rb0 = b * (5 * _S)

    def rbb_src(ckl):
        return rbb.at[pl.ds(rb0 + ckl * (5 * _CHUNK), 5 * _CHUNK)]

    pltpu.async_copy(rbb_src(0), rbbuf.at[pl.ds(0, 5 * _CHUNK)], sem_rb)

    def src_pos(ck):
        g = ck // _NCHUNK
        return (b * _C + ch0 + g * _NCH) * _S + (ck % _NCHUNK) * _CHUNK

    for j in range(_NCH):
        pltpu.async_copy(feats.at[pl.ds(src_pos(0) + j * _S, _CHUNK)],
                         outbuf.at[0, j], sem_fi)

    def chunk_body(ck, carry):
        g = ck // _NCHUNK
        ckl = ck % _NCHUNK
        oslot = ck % 3
        rslot = ck % 2
        c0 = ch0 + g * _NCH
        base = ckl * _CHUNK
        p0 = (b * _C + c0) * _S + base

        @pl.when(ck + 1 < _TOTCHUNK)
        def _prefetch_rbb():
            nxt = (ck + 1) % _NCHUNK
            pltpu.async_copy(rbb_src(nxt),
                             rbbuf.at[pl.ds((1 - rslot) * (5 * _CHUNK), 5 * _CHUNK)],
                             sem_rb)

        @pl.when(ckl == 0)
        def _load_planes():
            for j in range(_NCH):
                pltpu.async_copy(
                    feats.at[pl.ds((b * _C + c0 + j) * _S, _S)],
                    planes[j], sem_pl)
            for j in range(_NCH):
                pltpu.make_async_copy(
                    feats.at[pl.ds((b * _C + c0 + j) * _S, _S)],
                    planes[j], sem_pl).wait()

        @pl.when(ck >= 2)
        def _drain_out():
            for j in range(_NCH):
                pltpu.make_async_copy(
                    outbuf.at[(ck + 1) % 3, j],
                    out.at[pl.ds(src_pos(ck - 2) + j * _S, _CHUNK)],
                    sem_out).wait()

        @pl.when(ck + 1 < _TOTCHUNK)
        def _prefill_next():
            for j in range(_NCH):
                pltpu.async_copy(
                    feats.at[pl.ds(src_pos(ck + 1) + j * _S, _CHUNK)],
                    outbuf.at[(ck + 1) % 3, j], sem_fi)

        for j in range(_NCH):
            pltpu.make_async_copy(
                feats.at[pl.ds(p0 + j * _S, _CHUNK)],
                outbuf.at[oslot, j], sem_fi).wait()

        pltpu.make_async_copy(rbb_src(ckl),
                              rbbuf.at[pl.ds(rslot * (5 * _CHUNK), 5 * _CHUNK)],
                              sem_rb).wait()

        lane5 = lax.iota(jnp.int32, 16) * 5

        @plsc.parallel_loop(0, _CHUNK, step=16, unroll=1)
        def step_body(off):
            sl = pl.ds(base + off, 16)
            ipy = lane5 + (off * 5 + rslot * (5 * _CHUNK))
            y = plsc.load_gather(rbbuf, [ipy]) * _SCALE
            x = plsc.load_gather(rbbuf, [ipy + 1]) * _SCALE
            yl0 = y.astype(jnp.int32)
            xl0 = x.astype(jnp.int32)
            ly = y - yl0.astype(jnp.float32)
            lx = x - xl0.astype(jnp.float32)
            dx = jnp.minimum(xl0 + 1, _W - 1) - xl0
            dyo = (jnp.minimum(yl0 + 1, _H - 1) - yl0) * _W
            i1 = yl0 * _W + xl0
            i2 = i1 + dx
            i3 = i1 + dyo
            i4 = i3 + dx
            hy = 1.0 - ly
            hx = 1.0 - lx
            w1 = hy * hx
            w2 = hy * lx
            w3 = ly * hx
            w4 = ly * lx
            for j in range(_NCH):
                v1 = plsc.load_gather(planes[j], [i1])
                v2 = plsc.load_gather(planes[j], [i2])
                v3 = plsc.load_gather(planes[j], [i3])
                v4 = plsc.load_gather(planes[j], [i4])
                val = w1 * v1 + w2 * v2 + w3 * v3 + w4 * v4
                plsc.addupdate(outbuf.at[oslot, j, pl.ds(off, 16)], val)

        for j in range(_NCH):
            pltpu.async_copy(outbuf.at[oslot, j],
                             out.at[pl.ds(p0 + j * _S, _CHUNK)], sem_out)
        return carry

    lax.fori_loop(0, _TOTCHUNK, chunk_body, 0, unroll=False)

    for ck in (_TOTCHUNK - 2, _TOTCHUNK - 1):
        for j in range(_NCH):
            pltpu.make_async_copy(outbuf.at[ck % 3, j],
                                  out.at[pl.ds(src_pos(ck) + j * _S, _CHUNK)],
                                  sem_out).wait()


def kernel(features, best_rbboxes):
    f = features.reshape(_B * _C * _S)
    rbb = best_rbboxes.reshape(_B * 5 * _S)
    out = _fr_kernel(f, rbb)
    return out.reshape(_B, _C, _H, _W)


# speedup vs baseline: 1.0105x; 1.0105x over previous
import functools

import jax
import jax.numpy as jnp
from jax import lax
from jax.experimental import pallas as pl
from jax.experimental.pallas import tpu as pltpu
from jax.experimental.pallas import tpu_sc as plsc

_B, _C, _H, _W = 4, 256, 128, 128
_S = _H * _W
_SCALE = 0.125
_NW = 32
_WPB = _NW // _B
_CPW = _C // _WPB
_NCH = 4
_NGRP = _CPW // _NCH
_CHUNK = 2048
_NCHUNK = _S // _CHUNK
_TOTCHUNK = _NGRP * _NCHUNK

_mesh = plsc.VectorSubcoreMesh(core_axis_name="c", subcore_axis_name="s")


@functools.partial(
    pl.kernel,
    out_type=jax.ShapeDtypeStruct((_B * _C * _S,), jnp.float32),
    mesh=_mesh,
    compiler_params=pltpu.CompilerParams(needs_layout_passes=False),
    scratch_types=[
        [pltpu.VMEM((_S,), jnp.float32) for _ in range(_NCH)],
        pltpu.VMEM((2 * 5 * _CHUNK,), jnp.float32),
        pltpu.VMEM((3, _NCH, _CHUNK), jnp.float32),
        pltpu.SemaphoreType.DMA,
        pltpu.SemaphoreType.DMA,
        pltpu.SemaphoreType.DMA,
        pltpu.SemaphoreType.DMA,
    ],
)
def _fr_kernel(feats, rbb, out, planes, rbbuf, outbuf, sem_pl, sem_rb,
               sem_out, sem_fi):
    cid = lax.axis_index("c")
    sid = lax.axis_index("s")
    wid = sid * 2 + cid
    b = wid // _WPB
    ch0 = (wid % _WPB) * _CPW

    rb0 = b * (5 * _S)

    def rbb_src(ckl):
        return rbb.at[pl.ds(rb0 + ckl * (5 * _CHUNK), 5 * _CHUNK)]

    pltpu.async_copy(rbb_src(0), rbbuf.at[pl.ds(0, 5 * _CHUNK)], sem_rb)

    def src_pos(ck):
        g = ck // _NCHUNK
        return (b * _C + ch0 + g * _NCH) * _S + (ck % _NCHUNK) * _CHUNK

    for j in range(_NCH):
        pltpu.async_copy(feats.at[pl.ds(src_pos(0) + j * _S, _CHUNK)],
                         outbuf.at[0, j], sem_fi)

    def chunk_body(ck, carry):
        g = ck // _NCHUNK
        ckl = ck % _NCHUNK
        oslot = ck % 3
        rslot = ck % 2
        c0 = ch0 + g * _NCH
        base = ckl * _CHUNK
        p0 = (b * _C + c0) * _S + base

        @pl.when(ck + 1 < _TOTCHUNK)
        def _prefetch_rbb():
            nxt = (ck + 1) % _NCHUNK
            pltpu.async_copy(rbb_src(nxt),
                             rbbuf.at[pl.ds((1 - rslot) * (5 * _CHUNK),
                                            5 * _CHUNK)],
                             sem_rb)

        @pl.when(ckl == 0)
        def _load_planes():
            for j in range(_NCH):
                pltpu.async_copy(
                    feats.at[pl.ds((b * _C + c0 + j) * _S, _S)],
                    planes[j], sem_pl)
            for j in range(_NCH):
                pltpu.make_async_copy(
                    feats.at[pl.ds((b * _C + c0 + j) * _S, _S)],
                    planes[j], sem_pl).wait()

        @pl.when(ck >= 2)
        def _drain_out():
            for j in range(_NCH):
                pltpu.make_async_copy(
                    outbuf.at[(ck + 1) % 3, j],
                    out.at[pl.ds(src_pos(ck - 2) + j * _S, _CHUNK)],
                    sem_out).wait()

        @pl.when(ck + 1 < _TOTCHUNK)
        def _prefill_next():
            for j in range(_NCH):
                pltpu.async_copy(
                    feats.at[pl.ds(src_pos(ck + 1) + j * _S, _CHUNK)],
                    outbuf.at[(ck + 1) % 3, j], sem_fi)

        for j in range(_NCH):
            pltpu.make_async_copy(
                feats.at[pl.ds(p0 + j * _S, _CHUNK)],
                outbuf.at[oslot, j], sem_fi).wait()

        pltpu.make_async_copy(rbb_src(ckl),
                              rbbuf.at[pl.ds(rslot * (5 * _CHUNK),
                                             5 * _CHUNK)],
                              sem_rb).wait()

        lane5 = lax.iota(jnp.int32, 16) * 5

        @plsc.parallel_loop(0, _CHUNK, step=16, unroll=1)
        def step_body(off):
            ipy = lane5 + (off * 5 + rslot * (5 * _CHUNK))
            py = plsc.load_gather(rbbuf, [ipy]) * _SCALE
            px = plsc.load_gather(rbbuf, [ipy + 1]) * _SCALE
            y = jnp.maximum(py, 0.0)
            x = jnp.maximum(px, 0.0)
            yl0 = y.astype(jnp.int32)
            xl0 = x.astype(jnp.int32)
            ycond = yl0 >= _H - 1
            xcond = xl0 >= _W - 1
            y_low = jnp.where(ycond, _H - 1, yl0)
            x_low = jnp.where(xcond, _W - 1, xl0)
            ly = jnp.where(ycond, 0.0, y - yl0.astype(jnp.float32))
            lx = jnp.where(xcond, 0.0, x - xl0.astype(jnp.float32))
            dx = jnp.where(xcond, 0, 1)
            dyo = jnp.where(ycond, 0, _W)
            i1 = y_low * _W + x_low
            i2 = i1 + dx
            i3 = i1 + dyo
            i4 = i3 + dx
            hy = 1.0 - ly
            hx = 1.0 - lx
            w1 = hy * hx
            w2 = hy * lx
            w3 = ly * hx
            w4 = ly * lx
            for j in range(_NCH):
                v1 = plsc.load_gather(planes[j], [i1])
                v2 = plsc.load_gather(planes[j], [i2])
                v3 = plsc.load_gather(planes[j], [i3])
                v4 = plsc.load_gather(planes[j], [i4])
                val = w1 * v1 + w2 * v2 + w3 * v3 + w4 * v4
                plsc.addupdate(outbuf.at[oslot, j, pl.ds(off, 16)], val)

        for j in range(_NCH):
            pltpu.async_copy(outbuf.at[oslot, j],
                             out.at[pl.ds(p0 + j * _S, _CHUNK)], sem_out)
        return carry

    lax.fori_loop(0, _TOTCHUNK, chunk_body, 0, unroll=False)

    for ck in (_TOTCHUNK - 2, _TOTCHUNK - 1):
        for j in range(_NCH):
            pltpu.make_async_copy(outbuf.at[ck % 3, j],
                                  out.at[pl.ds(src_pos(ck) + j * _S, _CHUNK)],
                                  sem_out).wait()


def kernel(features, best_rbboxes):
    f = features.reshape(_B * _C * _S)
    rbb = best_rbboxes.reshape(_B * 5 * _S)
    out = _fr_kernel(f, rbb)
    return out.reshape(_B, _C, _H, _W)
